# Initial kernel scaffold; baseline (speedup 1.0000x reference)
import jax
import jax.numpy as jnp
from jax.experimental import pallas as pl


def kernel(x, adj1_0, adj1_1, adj1_2, adj2_0, adj2_1, adj2_2, W1, W2, r1_G1, r1_G2, r1_B1, r1_B2, r1_r, r2_G1, r2_G2, r2_B1, r2_B2, r2_r, g1_W, g2_W):
    raise NotImplementedError("write your pallas kernel here")


# speedup vs baseline: 1.1984x; 1.1984x over previous
import functools

import jax
import jax.numpy as jnp
from jax import lax
from jax.experimental import pallas as pl
from jax.experimental.pallas import tpu as pltpu
from jax.experimental.pallas import tpu_sc as plsc

_NC = 2
_NS = 16
_LN = 16
_NW = _NC * _NS

_N = 10000
_D = 128
_K = 16
_NPAD = 10240
_PER_W = _NPAD // _NW
_CH = 8
_NCHUNK = _PER_W // _CH
_IDXC = _CH * _K
_DC = _D // _LN


def _gather_reduce_body(table_hbm, idx_hbm, w_hbm, out_hbm,
                        idx_v, w_v, rows_v, out_v, sem):
    wid = lax.axis_index("s") * _NC + lax.axis_index("c")
    base = wid * _PER_W

    def chunk_body(c, carry):
        node0 = base + c * _CH
        e0 = node0 * _K
        pltpu.sync_copy(idx_hbm.at[pl.ds(e0, _IDXC)], idx_v)
        pltpu.sync_copy(w_hbm.at[pl.ds(e0, _IDXC)], w_v)
        pltpu.async_copy(table_hbm.at[idx_v], rows_v, sem).wait()

        def node_body(i, carry2):
            r0 = i * _K
            w_vec = w_v[pl.ds(r0, _K)]
            accs = [jnp.zeros((_LN,), jnp.float32) for _ in range(_DC)]
            for k in range(_K):
                wk = w_vec[k]
                for dci in range(_DC):
                    accs[dci] = accs[dci] + wk * rows_v[r0 + k, pl.ds(dci * _LN, _LN)]
            for dci in range(_DC):
                out_v[i, pl.ds(dci * _LN, _LN)] = accs[dci] * (1.0 / _K)
            return carry2

        lax.fori_loop(0, _CH, node_body, 0)
        pltpu.sync_copy(out_v, out_hbm.at[pl.ds(node0, _CH)])
        return carry

    lax.fori_loop(0, _NCHUNK, chunk_body, 0)


def _gather_reduce(table, idx_flat, w_flat):
    mesh = plsc.VectorSubcoreMesh(core_axis_name="c", subcore_axis_name="s")
    f = functools.partial(
        pl.kernel,
        mesh=mesh,
        out_type=jax.ShapeDtypeStruct((_NPAD, _D), jnp.float32),
        scratch_types=[
            pltpu.VMEM((_IDXC,), jnp.int32),
            pltpu.VMEM((_IDXC,), jnp.float32),
            pltpu.VMEM((_IDXC, _D), jnp.float32),
            pltpu.VMEM((_CH, _D), jnp.float32),
            pltpu.SemaphoreType.DMA,
        ],
    )(_gather_reduce_body)
    return f(table, idx_flat, w_flat)


def _mm_body(y_ref, w_ref, o_ref, *, act):
    v = jnp.dot(y_ref[...], w_ref[...], preferred_element_type=jnp.float32)
    if act:
        v = jnp.where(v > 0.0, v, jnp.exp(jnp.minimum(v, 0.0)) - 1.0)
    o_ref[...] = v


def _mm(y, w, act):
    n, d = y.shape
    dout = w.shape[1]
    blk = 2048
    return pl.pallas_call(
        functools.partial(_mm_body, act=act),
        grid=(n // blk,),
        in_specs=[pl.BlockSpec((blk, d), lambda i: (i, 0)),
                  pl.BlockSpec((d, dout), lambda i: (0, 0))],
        out_specs=pl.BlockSpec((blk, dout), lambda i: (i, 0)),
        out_shape=jax.ShapeDtypeStruct((n, dout), jnp.float32),
    )(y, w)


def _prep(a0, a2):
    idx = jnp.pad(a0[:, 1:].astype(jnp.int32), ((0, _NPAD - _N), (0, 0)))
    w = jnp.pad(a2[:, 1:].astype(jnp.float32), ((0, _NPAD - _N), (0, 0)))
    return idx.reshape(-1), w.reshape(-1)


def kernel(x, adj1_0, adj1_1, adj1_2, adj2_0, adj2_1, adj2_2, W1, W2,
           r1_G1, r1_G2, r1_B1, r1_B2, r1_r,
           r2_G1, r2_G2, r2_B1, r2_B2, r2_r,
           g1_W, g2_W):
    idx1, w1 = _prep(adj1_0, adj1_2)
    y1 = _gather_reduce(x, idx1, w1)
    h1 = _mm(y1, W1, act=True)

    idx2, w2 = _prep(adj2_0, adj2_2)
    y2 = _gather_reduce(h1, idx2, w2)
    h2 = _mm(y2, W2, act=False)
    return h2[:_N]


# speedup vs baseline: 1.4517x; 1.2114x over previous
import functools

import jax
import jax.numpy as jnp
from jax import lax
from jax.experimental import pallas as pl
from jax.experimental.pallas import tpu as pltpu
from jax.experimental.pallas import tpu_sc as plsc

_NC = 2
_NS = 16
_LN = 16
_NW = _NC * _NS

_N = 10000
_D = 128
_K = 16
_NPAD = 10240
_PER_W = _NPAD // _NW
_CH = 8
_NCHUNK = _PER_W // _CH
_IDXC = _CH * _K
_DC = _D // _LN


def _gather_reduce_body(table_hbm, idx_hbm, w_hbm, out_hbm,
                        idx_v, w_v, rows0, rows1, out_v, sem0, sem1):
    wid = lax.axis_index("s") * _NC + lax.axis_index("c")

    pltpu.sync_copy(idx_hbm.at[pl.ds(wid * _NCHUNK, _NCHUNK)], idx_v)
    pltpu.sync_copy(w_hbm.at[pl.ds(wid * _NCHUNK, _NCHUNK)], w_v)

    bufs = ((rows0, sem0), (rows1, sem1))

    def start(c, rows, sem):
        pltpu.async_copy(table_hbm.at[idx_v.at[c]], rows, sem)

    def wait(c, rows, sem):
        pltpu.make_async_copy(table_hbm.at[idx_v.at[c]], rows, sem).wait()

    def compute(c, rows):
        def node_body(i, carry):
            w_vec = w_v[c, pl.ds(i * _K, _K)]
            r0 = i * _K
            accs = [jnp.zeros((_LN,), jnp.float32) for _ in range(_DC)]
            for k in range(_K):
                wk = w_vec[k]
                for dci in range(_DC):
                    accs[dci] = accs[dci] + wk * rows[r0 + k, pl.ds(dci * _LN, _LN)]
            for dci in range(_DC):
                out_v[c * _CH + i, pl.ds(dci * _LN, _LN)] = accs[dci] * (1.0 / _K)
            return carry

        lax.fori_loop(0, _CH, node_body, 0)

    start(0, *bufs[0])
    start(1, *bufs[1])

    def pair_body(c2, carry):
        for p in range(2):
            rows, sem = bufs[p]
            c = c2 * 2 + p
            wait(c, rows, sem)
            compute(c, rows)

            @pl.when(c + 2 < _NCHUNK)
            def _():
                start(c + 2, rows, sem)
        return carry

    lax.fori_loop(0, _NCHUNK // 2, pair_body, 0)
    pltpu.sync_copy(out_v, out_hbm.at[pl.ds(wid * _PER_W, _PER_W)])


def _gather_reduce(table, idx2d, w2d):
    mesh = plsc.VectorSubcoreMesh(core_axis_name="c", subcore_axis_name="s")
    f = functools.partial(
        pl.kernel,
        mesh=mesh,
        out_type=jax.ShapeDtypeStruct((_NPAD, _D), jnp.float32),
        scratch_types=[
            pltpu.VMEM((_NCHUNK, _IDXC), jnp.int32),
            pltpu.VMEM((_NCHUNK, _IDXC), jnp.float32),
            pltpu.VMEM((_IDXC, _D), jnp.float32),
            pltpu.VMEM((_IDXC, _D), jnp.float32),
            pltpu.VMEM((_PER_W, _D), jnp.float32),
            pltpu.SemaphoreType.DMA,
            pltpu.SemaphoreType.DMA,
        ],
    )(_gather_reduce_body)
    return f(table, idx2d, w2d)


def _mm_body(y_ref, w_ref, o_ref, *, act):
    v = jnp.dot(y_ref[...], w_ref[...], preferred_element_type=jnp.float32)
    if act:
        v = jnp.where(v > 0.0, v, jnp.exp(jnp.minimum(v, 0.0)) - 1.0)
    o_ref[...] = v


def _mm(y, w, act):
    n, d = y.shape
    dout = w.shape[1]
    blk = 2048
    return pl.pallas_call(
        functools.partial(_mm_body, act=act),
        grid=(n // blk,),
        in_specs=[pl.BlockSpec((blk, d), lambda i: (i, 0)),
                  pl.BlockSpec((d, dout), lambda i: (0, 0))],
        out_specs=pl.BlockSpec((blk, dout), lambda i: (i, 0)),
        out_shape=jax.ShapeDtypeStruct((n, dout), jnp.float32),
    )(y, w)


def _prep(a0, a2):
    idx = jnp.pad(a0[:, 1:].astype(jnp.int32), ((0, _NPAD - _N), (0, 0)))
    w = jnp.pad(a2[:, 1:].astype(jnp.float32), ((0, _NPAD - _N), (0, 0)))
    return (idx.reshape(_NW * _NCHUNK, _IDXC),
            w.reshape(_NW * _NCHUNK, _IDXC))


def kernel(x, adj1_0, adj1_1, adj1_2, adj2_0, adj2_1, adj2_2, W1, W2,
           r1_G1, r1_G2, r1_B1, r1_B2, r1_r,
           r2_G1, r2_G2, r2_B1, r2_B2, r2_r,
           g1_W, g2_W):
    idx1, w1 = _prep(adj1_0, adj1_2)
    y1 = _gather_reduce(x, idx1, w1)
    h1 = _mm(y1, W1, act=True)

    idx2, w2 = _prep(adj2_0, adj2_2)
    y2 = _gather_reduce(h1, idx2, w2)
    h2 = _mm(y2, W2, act=False)
    return h2[:_N]


# speedup vs baseline: 5.6953x; 3.9232x over previous
import functools

import jax
import jax.numpy as jnp
from jax import lax
from jax.experimental import pallas as pl
from jax.experimental.pallas import tpu as pltpu
from jax.experimental.pallas import tpu_sc as plsc

_NC = 2
_NS = 16
_LN = 16
_NW = _NC * _NS

_N = 10000
_D = 128
_K = 16
_NPAD = 10240
_PER_W = _NPAD // _NW
_CH = 4
_NCHUNK = _PER_W // _CH
_IDXC = _CH * _K
_DC = _D // _LN
_RPS = _NPAD // _NS


def _gather_reduce_body(table_hbm, idx_hbm, w_hbm, out_hbm,
                        idx_v, w_v, rows0, rows1, ob0, ob1, shared,
                        sem0, sem1, osem0, osem1):
    sid = lax.axis_index("s")
    wid = sid * _NC + lax.axis_index("c")
    base = wid * _PER_W

    pltpu.sync_copy(table_hbm.at[pl.ds(sid * _RPS, _RPS)],
                    shared.at[pl.ds(sid * _RPS, _RPS)])

    pltpu.sync_copy(idx_hbm.at[pl.ds(wid * _NCHUNK, _NCHUNK)], idx_v)
    pltpu.sync_copy(w_hbm.at[pl.ds(wid * _NCHUNK, _NCHUNK)], w_v)

    plsc.subcore_barrier()

    bufs = ((rows0, sem0, ob0, osem0), (rows1, sem1, ob1, osem1))

    def start(c, rows, sem):
        pltpu.async_copy(shared.at[idx_v.at[c]], rows, sem)

    def wait(c, rows, sem):
        pltpu.make_async_copy(shared.at[idx_v.at[c]], rows, sem).wait()

    def out_slice(c):
        return out_hbm.at[pl.ds(base + c * _CH, _CH)]

    def compute(c, rows, ob):
        def node_body(i, carry):
            w_vec = w_v[c, pl.ds(i * _K, _K)]
            r0 = i * _K
            accs = [jnp.zeros((_LN,), jnp.float32) for _ in range(_DC)]
            for k in range(_K):
                wk = w_vec[k]
                for dci in range(_DC):
                    accs[dci] = accs[dci] + wk * rows[r0 + k, pl.ds(dci * _LN, _LN)]
            for dci in range(_DC):
                ob[i, pl.ds(dci * _LN, _LN)] = accs[dci] * (1.0 / _K)
            return carry

        lax.fori_loop(0, _CH, node_body, 0)

    start(0, rows0, sem0)
    start(1, rows1, sem1)

    def pair_body(c2, carry):
        for p in range(2):
            rows, sem, ob, osem = bufs[p]
            c = c2 * 2 + p
            wait(c, rows, sem)

            @pl.when(c >= 2)
            def _():
                pltpu.make_async_copy(ob, out_slice(c), osem).wait()

            compute(c, rows, ob)
            pltpu.async_copy(ob, out_slice(c), osem)

            @pl.when(c + 2 < _NCHUNK)
            def _():
                start(c + 2, rows, sem)
        return carry

    lax.fori_loop(0, _NCHUNK // 2, pair_body, 0)
    pltpu.make_async_copy(ob0, out_slice(_NCHUNK - 2), osem0).wait()
    pltpu.make_async_copy(ob1, out_slice(_NCHUNK - 1), osem1).wait()


def _gather_reduce(table, idx2d, w2d):
    mesh = plsc.VectorSubcoreMesh(core_axis_name="c", subcore_axis_name="s")
    f = functools.partial(
        pl.kernel,
        mesh=mesh,
        out_type=jax.ShapeDtypeStruct((_NPAD, _D), jnp.float32),
        scratch_types=[
            pltpu.VMEM((_NCHUNK, _IDXC), jnp.int32),
            pltpu.VMEM((_NCHUNK, _IDXC), jnp.float32),
            pltpu.VMEM((_IDXC, _D), jnp.float32),
            pltpu.VMEM((_IDXC, _D), jnp.float32),
            pltpu.VMEM((_CH, _D), jnp.float32),
            pltpu.VMEM((_CH, _D), jnp.float32),
            pltpu.VMEM_SHARED((_NPAD, _D), jnp.float32),
            pltpu.SemaphoreType.DMA,
            pltpu.SemaphoreType.DMA,
            pltpu.SemaphoreType.DMA,
            pltpu.SemaphoreType.DMA,
        ],
    )(_gather_reduce_body)
    return f(table, idx2d, w2d)


def _mm_body(y_ref, w_ref, o_ref, *, act):
    v = jnp.dot(y_ref[...], w_ref[...], preferred_element_type=jnp.float32)
    if act:
        v = jnp.where(v > 0.0, v, jnp.exp(jnp.minimum(v, 0.0)) - 1.0)
    o_ref[...] = v


def _mm(y, w, act):
    n, d = y.shape
    dout = w.shape[1]
    blk = 2048
    return pl.pallas_call(
        functools.partial(_mm_body, act=act),
        grid=(n // blk,),
        in_specs=[pl.BlockSpec((blk, d), lambda i: (i, 0)),
                  pl.BlockSpec((d, dout), lambda i: (0, 0))],
        out_specs=pl.BlockSpec((blk, dout), lambda i: (i, 0)),
        out_shape=jax.ShapeDtypeStruct((n, dout), jnp.float32),
    )(y, w)


def _prep(a0, a2):
    idx = jnp.pad(a0[:, 1:].astype(jnp.int32), ((0, _NPAD - _N), (0, 0)))
    w = jnp.pad(a2[:, 1:].astype(jnp.float32), ((0, _NPAD - _N), (0, 0)))
    return (idx.reshape(_NW * _NCHUNK, _IDXC),
            w.reshape(_NW * _NCHUNK, _IDXC))


def kernel(x, adj1_0, adj1_1, adj1_2, adj2_0, adj2_1, adj2_2, W1, W2,
           r1_G1, r1_G2, r1_B1, r1_B2, r1_r,
           r2_G1, r2_G2, r2_B1, r2_B2, r2_r,
           g1_W, g2_W):
    idx1, w1 = _prep(adj1_0, adj1_2)
    xp = jnp.pad(x, ((0, _NPAD - _N), (0, 0)))
    y1 = _gather_reduce(xp, idx1, w1)
    h1 = _mm(y1, W1, act=True)

    idx2, w2 = _prep(adj2_0, adj2_2)
    y2 = _gather_reduce(h1, idx2, w2)
    h2 = _mm(y2, W2, act=False)
    return h2[:_N]


# speedup vs baseline: 6.2254x; 1.0931x over previous
import functools

import jax
import jax.numpy as jnp
from jax import lax
from jax.experimental import pallas as pl
from jax.experimental.pallas import tpu as pltpu
from jax.experimental.pallas import tpu_sc as plsc

_NC = 2
_NS = 16
_LN = 16
_NW = _NC * _NS

_N = 10000
_K = 16
_NPAD = 10240
_PER_W = _NPAD // _NW
_CH = 4
_NCHUNK = _PER_W // _CH
_IDXC = _CH * _K
_RPS = _NPAD // _NS


def _make_gr_body(d, act):
    dc = d // _LN

    def body(table_hbm, idx_hbm, w_hbm, out_hbm,
             idx_v, w_v, rows0, rows1, ob0, ob1, shared,
             sem0, sem1, osem0, osem1):
        sid = lax.axis_index("s")
        wid = sid * _NC + lax.axis_index("c")
        base = wid * _PER_W

        pltpu.sync_copy(table_hbm.at[pl.ds(sid * _RPS, _RPS)],
                        shared.at[pl.ds(sid * _RPS, _RPS)])

        pltpu.sync_copy(idx_hbm.at[pl.ds(wid * _NCHUNK, _NCHUNK)], idx_v)
        pltpu.sync_copy(w_hbm.at[pl.ds(wid * _NCHUNK, _NCHUNK)], w_v)

        plsc.subcore_barrier()

        bufs = ((rows0, sem0, ob0, osem0), (rows1, sem1, ob1, osem1))

        def start(c, rows, sem):
            pltpu.async_copy(shared.at[idx_v.at[c]], rows, sem)

        def wait(c, rows, sem):
            pltpu.make_async_copy(shared.at[idx_v.at[c]], rows, sem).wait()

        def out_slice(c):
            return out_hbm.at[pl.ds(base + c * _CH, _CH)]

        def compute(c, rows, ob):
            def node_body(i, carry):
                w_vec = w_v[c, pl.ds(i * _K, _K)]
                r0 = i * _K
                accs = [jnp.zeros((_LN,), jnp.float32) for _ in range(dc)]
                for k in range(_K):
                    wk = w_vec[k]
                    for dci in range(dc):
                        accs[dci] = accs[dci] + wk * rows[r0 + k, pl.ds(dci * _LN, _LN)]
                for dci in range(dc):
                    v = accs[dci] * (1.0 / _K)
                    if act:
                        v = jnp.where(v > 0.0, v,
                                      jnp.exp(jnp.minimum(v, 0.0)) - 1.0)
                    ob[i, pl.ds(dci * _LN, _LN)] = v
                return carry

            lax.fori_loop(0, _CH, node_body, 0)

        start(0, rows0, sem0)
        start(1, rows1, sem1)

        def pair_body(c2, carry):
            for p in range(2):
                rows, sem, ob, osem = bufs[p]
                c = c2 * 2 + p
                wait(c, rows, sem)

                @pl.when(c >= 2)
                def _():
                    pltpu.make_async_copy(ob, out_slice(c), osem).wait()

                compute(c, rows, ob)
                pltpu.async_copy(ob, out_slice(c), osem)

                @pl.when(c + 2 < _NCHUNK)
                def _():
                    start(c + 2, rows, sem)
            return carry

        lax.fori_loop(0, _NCHUNK // 2, pair_body, 0)
        pltpu.make_async_copy(ob0, out_slice(_NCHUNK - 2), osem0).wait()
        pltpu.make_async_copy(ob1, out_slice(_NCHUNK - 1), osem1).wait()

    return body


def _gather_reduce(table, idx2d, w2d, act):
    d = table.shape[1]
    mesh = plsc.VectorSubcoreMesh(core_axis_name="c", subcore_axis_name="s")
    f = functools.partial(
        pl.kernel,
        mesh=mesh,
        out_type=jax.ShapeDtypeStruct((_NPAD, d), jnp.float32),
        scratch_types=[
            pltpu.VMEM((_NCHUNK, _IDXC), jnp.int32),
            pltpu.VMEM((_NCHUNK, _IDXC), jnp.float32),
            pltpu.VMEM((_IDXC, d), jnp.float32),
            pltpu.VMEM((_IDXC, d), jnp.float32),
            pltpu.VMEM((_CH, d), jnp.float32),
            pltpu.VMEM((_CH, d), jnp.float32),
            pltpu.VMEM_SHARED((_NPAD, d), jnp.float32),
            pltpu.SemaphoreType.DMA,
            pltpu.SemaphoreType.DMA,
            pltpu.SemaphoreType.DMA,
            pltpu.SemaphoreType.DMA,
        ],
    )(_make_gr_body(d, act))
    return f(table, idx2d, w2d)


def _mm_body(y_ref, w_ref, o_ref):
    o_ref[...] = jnp.dot(y_ref[...], w_ref[...],
                         preferred_element_type=jnp.float32)


def _mm(y, w):
    n, d = y.shape
    dout = w.shape[1]
    blk = 2048
    return pl.pallas_call(
        _mm_body,
        grid=(n // blk,),
        in_specs=[pl.BlockSpec((blk, d), lambda i: (i, 0)),
                  pl.BlockSpec((d, dout), lambda i: (0, 0))],
        out_specs=pl.BlockSpec((blk, dout), lambda i: (i, 0)),
        out_shape=jax.ShapeDtypeStruct((n, dout), jnp.float32),
    )(y, w)


def _prep(a0, a2):
    idx = jnp.pad(a0[:, 1:].astype(jnp.int32), ((0, _NPAD - _N), (0, 0)))
    w = jnp.pad(a2[:, 1:].astype(jnp.float32), ((0, _NPAD - _N), (0, 0)))
    return (idx.reshape(_NW * _NCHUNK, _IDXC),
            w.reshape(_NW * _NCHUNK, _IDXC))


def kernel(x, adj1_0, adj1_1, adj1_2, adj2_0, adj2_1, adj2_2, W1, W2,
           r1_G1, r1_G2, r1_B1, r1_B2, r1_r,
           r2_G1, r2_G2, r2_B1, r2_B2, r2_r,
           g1_W, g2_W):
    idx1, w1 = _prep(adj1_0, adj1_2)
    idx2, w2 = _prep(adj2_0, adj2_2)
    xp = jnp.pad(x, ((0, _NPAD - _N), (0, 0)))

    xw1 = _mm(xp, W1)
    h1 = _gather_reduce(xw1, idx1, w1, act=True)
    g2 = _mm(h1, W2)
    h2 = _gather_reduce(g2, idx2, w2, act=False)
    return h2[:_N]
